# Initial kernel scaffold; baseline (speedup 1.0000x reference)
import jax
import jax.numpy as jnp
from jax.experimental import pallas as pl


def kernel(logits, labels):
    raise NotImplementedError("write your pallas kernel here")


# speedup vs baseline: 7.9043x; 7.9043x over previous
import jax
import jax.numpy as jnp
from jax.experimental import pallas as pl
from jax.experimental.pallas import tpu as pltpu

_TOL = 5
_PROX_REWARD = 2.0
_FAR_PENALTY = 1.5
_W0 = 0.1
_W12 = 5.0

_BB = 8


def _shift_fwd(x, d):
    z = jnp.zeros((x.shape[0], d), x.dtype)
    return jnp.concatenate([x[:, d:], z], axis=1)


def _shift_bwd(x, d):
    z = jnp.zeros((x.shape[0], d), x.dtype)
    return jnp.concatenate([z, x[:, :-d]], axis=1)


def _window_any(m):
    a = jnp.maximum(m, _shift_fwd(m, 1))
    b = jnp.maximum(a, _shift_fwd(a, 2))
    c = jnp.maximum(b, _shift_fwd(b, 2))
    d1 = jnp.maximum(c, _shift_bwd(c, 1))
    d2 = jnp.maximum(d1, _shift_bwd(d1, 2))
    return jnp.maximum(d2, _shift_bwd(d2, 2))


def _body(x0_ref, x1_ref, x2_ref, lab_ref, out_ref):
    x0 = x0_ref[...]
    x1 = x1_ref[...]
    x2 = x2_ref[...]
    lab = lab_ref[...]

    m12 = jnp.maximum(x1, x2)
    m = jnp.maximum(x0, m12)
    sumexp = jnp.exp(x0 - m) + jnp.exp(x1 - m) + jnp.exp(x2 - m)
    lse = jnp.log(sumexp) + m

    is0 = lab == 0
    is1 = lab == 1
    x_at_label = jnp.where(is0, x0, jnp.where(is1, x1, x2))
    w_at_label = jnp.where(is0, _W0, _W12)
    base = w_at_label * (lse - x_at_label)

    pred_sw = jnp.where(m12 > x0, 1.0, 0.0)
    true_sw = jnp.where(lab >= 1, 1.0, 0.0)
    pred_near = _window_any(pred_sw)
    true_near = _window_any(true_sw)

    has_true = jnp.max(true_sw, axis=1, keepdims=True)
    pen_row = jnp.sum(pred_sw * (1.0 - true_near), axis=1, keepdims=True)

    total = (jnp.sum(base, axis=(0, 1), keepdims=True)
             - _PROX_REWARD * jnp.sum(true_sw * pred_near, axis=(0, 1), keepdims=True)
             + _FAR_PENALTY * jnp.sum(pen_row * has_true, axis=(0, 1), keepdims=True))
    out_ref[...] = total[None]


def kernel(logits, labels):
    B, S, _C = logits.shape
    x0 = logits[:, :, 0]
    x1 = logits[:, :, 1]
    x2 = logits[:, :, 2]
    nb = B // _BB

    bs = pl.BlockSpec((_BB, S), lambda i: (i, 0))
    partials = pl.pallas_call(
        _body,
        grid=(nb,),
        in_specs=[bs, bs, bs, bs],
        out_specs=pl.BlockSpec((1, 1, 1), lambda i: (i, 0, 0)),
        out_shape=jax.ShapeDtypeStruct((nb, 1, 1), jnp.float32),
        compiler_params=pltpu.CompilerParams(
            dimension_semantics=("parallel",)),
    )(x0, x1, x2, labels)
    return jnp.sum(partials) / (B * S)
